# baseline (speedup 1.0000x reference)
import functools

import jax
import jax.numpy as jnp
from jax import lax
from jax.experimental import pallas as pl
from jax.experimental.pallas import tpu as pltpu, tpu_sc as plsc


def _build_sc_kernel(B, C, Tp, Hp, Wp, Tout, CP):
    info = plsc.get_sparse_core_info()
    NC, NS, LANES = info.num_cores, info.num_subcores, info.num_lanes
    NW = NC * NS
    L = Hp * Wp
    Wout = 2 * Wp
    n_tasks = B * Tp * Hp
    assert n_tasks % (2 * NW) == 0 and C % LANES == 0
    per_tile = n_tasks // NW
    blk = Wp * C
    row_out = Wout * CP
    slab = 2 * row_out

    mesh = plsc.VectorSubcoreMesh(core_axis_name="c", subcore_axis_name="s")

    @functools.partial(
        pl.kernel,
        mesh=mesh,
        out_type=jax.ShapeDtypeStruct((B * Tout * 2 * Hp * row_out,), jnp.float32),
        compiler_params=pltpu.CompilerParams(needs_layout_passes=False),
        scratch_types=[
            pltpu.VMEM((2 * blk,), jnp.float32),
            pltpu.VMEM((2 * blk,), jnp.int32),
            pltpu.VMEM((4 * blk,), jnp.int32),
            pltpu.VMEM((4 * slab,), jnp.float32),
            pltpu.SemaphoreType.DMA,
            pltpu.SemaphoreType.DMA,
            pltpu.SemaphoreType.DMA,
            pltpu.SemaphoreType.DMA,
        ],
    )
    def k(x_hbm, ot_hbm, is_hbm, out_hbm, xv, otv, isv, outv, siA, siB, soA, soB):
        wid = lax.axis_index("s") * NC + lax.axis_index("c")
        zero16 = jnp.zeros((LANES,), jnp.float32)
        si = (siA, siB)
        so = (soA, soB)

        def decomp(task):
            bp = task // Hp
            hp = task - bp * Hp
            b = bp // Tp
            p = bp - b * Tp
            return b, p, hp

        def in_srcs(task):
            task = jnp.minimum(task, n_tasks - 1)
            b, p, hp = decomp(task)
            xs = x_hbm.at[pl.ds(((b * Hp + hp) * Tp + p) * blk, blk)]
            os_ = ot_hbm.at[pl.ds(((b * Tp + p) * L + hp * Wp) * C, blk)]
            r0 = (((2 * p) * B + b) * Hp + hp) * blk
            bstep = B * Hp * blk
            is0 = is_hbm.at[pl.ds(r0, blk)]
            is1 = is_hbm.at[pl.ds(r0 + bstep, blk)]
            return xs, os_, is0, is1

        def in_dsts(ph):
            return (xv.at[pl.ds(ph * blk, blk)],
                    otv.at[pl.ds(ph * blk, blk)],
                    isv.at[pl.ds(2 * ph * blk, blk)],
                    isv.at[pl.ds((2 * ph + 1) * blk, blk)])

        def in_start(task, ph):
            for s, d in zip(in_srcs(task), in_dsts(ph)):
                pltpu.async_copy(s, d, si[ph])

        def in_wait(ph):
            for s, d in zip(in_srcs(0), in_dsts(ph)):
                pltpu.make_async_copy(s, d, si[ph]).wait()

        def out_parts(task, ph):
            b, p, hp = decomp(task)
            base = ((b * Tout + 2 * p) * Hp + hp) * slab
            hstep = Hp * slab
            return ((outv.at[pl.ds(2 * ph * slab, slab)],
                     out_hbm.at[pl.ds(base, slab)]),
                    (outv.at[pl.ds((2 * ph + 1) * slab, slab)],
                     out_hbm.at[pl.ds(base + hstep, slab)]))

        def out_start(task, ph):
            for s, d in out_parts(task, ph):
                pltpu.async_copy(s, d, so[ph])

        def out_wait(ph):
            for s, d in out_parts(0, ph):
                pltpu.make_async_copy(s, d, so[ph]).wait()

        def compute(task, ph):
            b, p, hp = decomp(task)

            def make_body(tt):
                obase = (2 * ph + tt) * slab

                def body(wp, _):
                    ubase = 2 * hp * Wout + 2 * wp
                    for cb in range(0, C, LANES):
                        val = xv[pl.ds(ph * blk + wp * C + cb, LANES)]
                        o = otv[pl.ds(ph * blk + wp * C + cb, LANES)]
                        s = isv[pl.ds((2 * ph + tt) * blk + wp * C + cb, LANES)]
                        mt = o == (2 * p + tt)
                        u = jnp.where(mt, s - ubase, -1)
                        for oh in (0, 1):
                            for ow in (0, 1):
                                m = u == (oh * Wout + ow)
                                res = jnp.where(m, val, zero16)
                                pos = obase + oh * row_out + (2 * wp + ow) * CP + cb
                                outv[pl.ds(pos, LANES)] = res
                    return 0
                return body

            lax.fori_loop(0, Wp, make_body(0), 0)
            lax.fori_loop(0, Wp, make_body(1), 0)

        first = wid * per_tile
        in_start(first, 0)

        def step(kk, _):
            t0 = first + 2 * kk
            in_start(t0 + 1, 1)
            in_wait(0)

            @pl.when(kk > 0)
            def _():
                out_wait(0)

            compute(t0, 0)
            out_start(t0, 0)
            in_start(t0 + 2, 0)
            in_wait(1)

            @pl.when(kk > 0)
            def _():
                out_wait(1)

            compute(t0 + 1, 1)
            out_start(t0 + 1, 1)
            return 0

        lax.fori_loop(0, per_tile // 2, step, 0)
        out_wait(0)
        out_wait(1)
        in_wait(0)

    return k


def kernel(x, inds_spatial, inds_temporal, siz):
    B, C, Tp, Hp, Wp = x.shape
    Tout = inds_spatial.shape[0]
    Hout, Wout = 2 * Hp, 2 * Wp
    CP = 128

    xP = jnp.transpose(x, (0, 3, 2, 4, 1)).reshape(-1)
    otP = jnp.transpose(inds_temporal, (1, 3, 0, 2)).reshape(-1)
    isP = jnp.transpose(inds_spatial, (0, 1, 3, 4, 2)).reshape(-1)

    k = _build_sc_kernel(B, C, Tp, Hp, Wp, Tout, CP)
    out_flat = k(xP, otP, isP)
    out5 = out_flat.reshape(B, Tout, Hout, Wout, CP)
    return jnp.transpose(out5, (0, 4, 1, 2, 3))[:, :C]


# speedup vs baseline: 1.0194x; 1.0194x over previous
import functools

import jax
import jax.numpy as jnp
from jax import lax
from jax.experimental import pallas as pl
from jax.experimental.pallas import tpu as pltpu, tpu_sc as plsc


def _build_sc_kernel(B, C, Tp, Hp, Wp, Tout, CP):
    info = plsc.get_sparse_core_info()
    NC, NS, LANES = info.num_cores, info.num_subcores, info.num_lanes
    NW = NC * NS
    L = Hp * Wp
    Wout = 2 * Wp
    n_tasks = B * Tp * Hp
    assert n_tasks % (2 * NW) == 0 and C % LANES == 0
    per_tile = n_tasks // NW
    blk = Wp * C
    row_out = Wout * CP
    slab = 2 * row_out

    mesh = plsc.VectorSubcoreMesh(core_axis_name="c", subcore_axis_name="s")

    @functools.partial(
        pl.kernel,
        mesh=mesh,
        out_type=jax.ShapeDtypeStruct((B * Tout * 2 * Hp * row_out,), jnp.float32),
        compiler_params=pltpu.CompilerParams(needs_layout_passes=False),
        scratch_types=[
            pltpu.VMEM((2 * blk,), jnp.float32),
            pltpu.VMEM((2 * blk,), jnp.int32),
            pltpu.VMEM((4 * blk,), jnp.int32),
            pltpu.VMEM((4 * slab,), jnp.float32),
            pltpu.SemaphoreType.DMA,
            pltpu.SemaphoreType.DMA,
            pltpu.SemaphoreType.DMA,
            pltpu.SemaphoreType.DMA,
        ],
    )
    def k(x_hbm, ot_hbm, is_hbm, out_hbm, xv, otv, isv, outv, siA, siB, soA, soB):
        wid = lax.axis_index("s") * NC + lax.axis_index("c")
        zero16 = jnp.zeros((LANES,), jnp.float32)
        si = (siA, siB)
        so = (soA, soB)

        def decomp(task):
            bp = task // Hp
            hp = task - bp * Hp
            b = bp // Tp
            p = bp - b * Tp
            return b, p, hp

        def in_srcs(task):
            task = jnp.minimum(task, n_tasks - 1)
            b, p, hp = decomp(task)
            xs = x_hbm.at[pl.ds(((b * Hp + hp) * Tp + p) * blk, blk)]
            os_ = ot_hbm.at[pl.ds(((b * Tp + p) * L + hp * Wp) * C, blk)]
            r0 = (((2 * p) * B + b) * Hp + hp) * blk
            bstep = B * Hp * blk
            is0 = is_hbm.at[pl.ds(r0, blk)]
            is1 = is_hbm.at[pl.ds(r0 + bstep, blk)]
            return xs, os_, is0, is1

        def in_dsts(ph):
            return (xv.at[pl.ds(ph * blk, blk)],
                    otv.at[pl.ds(ph * blk, blk)],
                    isv.at[pl.ds(2 * ph * blk, blk)],
                    isv.at[pl.ds((2 * ph + 1) * blk, blk)])

        def in_start(task, ph):
            for s, d in zip(in_srcs(task), in_dsts(ph)):
                pltpu.async_copy(s, d, si[ph])

        def in_wait(ph):
            for s, d in zip(in_srcs(0), in_dsts(ph)):
                pltpu.make_async_copy(s, d, si[ph]).wait()

        def out_parts(task, ph):
            b, p, hp = decomp(task)
            base = ((b * Tout + 2 * p) * Hp + hp) * slab
            hstep = Hp * slab
            return ((outv.at[pl.ds(2 * ph * slab, slab)],
                     out_hbm.at[pl.ds(base, slab)]),
                    (outv.at[pl.ds((2 * ph + 1) * slab, slab)],
                     out_hbm.at[pl.ds(base + hstep, slab)]))

        def out_start(task, ph):
            for s, d in out_parts(task, ph):
                pltpu.async_copy(s, d, so[ph])

        def out_wait(ph):
            for s, d in out_parts(0, ph):
                pltpu.make_async_copy(s, d, so[ph]).wait()

        def compute(task, ph):
            b, p, hp = decomp(task)

            def make_body(tt):
                obase = (2 * ph + tt) * slab

                def body(wp, _):
                    ubase = 2 * hp * Wout + 2 * wp
                    for cb in range(0, C, LANES):
                        val = xv[pl.ds(ph * blk + wp * C + cb, LANES)]
                        o = otv[pl.ds(ph * blk + wp * C + cb, LANES)]
                        s = isv[pl.ds((2 * ph + tt) * blk + wp * C + cb, LANES)]
                        u = s - ubase
                        mt = o == (2 * p + tt)
                        for oh in (0, 1):
                            for ow in (0, 1):
                                m = jnp.logical_and(u == (oh * Wout + ow), mt)
                                res = jnp.where(m, val, zero16)
                                pos = obase + oh * row_out + (2 * wp + ow) * CP + cb
                                outv[pl.ds(pos, LANES)] = res
                    return 0
                return body

            lax.fori_loop(0, Wp, make_body(0), 0)
            lax.fori_loop(0, Wp, make_body(1), 0)

        first = wid * per_tile
        in_start(first, 0)

        def step(kk, _):
            t0 = first + 2 * kk
            in_start(t0 + 1, 1)
            in_wait(0)

            @pl.when(kk > 0)
            def _():
                out_wait(0)

            compute(t0, 0)
            out_start(t0, 0)
            in_start(t0 + 2, 0)
            in_wait(1)

            @pl.when(kk > 0)
            def _():
                out_wait(1)

            compute(t0 + 1, 1)
            out_start(t0 + 1, 1)
            return 0

        lax.fori_loop(0, per_tile // 2, step, 0)
        out_wait(0)
        out_wait(1)
        in_wait(0)

    return k


def kernel(x, inds_spatial, inds_temporal, siz):
    B, C, Tp, Hp, Wp = x.shape
    Tout = inds_spatial.shape[0]
    Hout, Wout = 2 * Hp, 2 * Wp
    CP = 128

    xP = jnp.transpose(x, (0, 3, 2, 4, 1)).reshape(-1)
    otP = jnp.transpose(inds_temporal, (1, 3, 0, 2)).reshape(-1)
    isP = jnp.transpose(inds_spatial, (0, 1, 3, 4, 2)).reshape(-1)

    k = _build_sc_kernel(B, C, Tp, Hp, Wp, Tout, CP)
    out_flat = k(xP, otP, isP)
    out5 = out_flat.reshape(B, Tout, Hout, Wout, CP)
    return jnp.transpose(out5, (0, 4, 1, 2, 3))[:, :C]
